# baseline (speedup 1.0000x reference)
import functools

import jax
import jax.numpy as jnp
from jax import lax
from jax.experimental import pallas as pl
from jax.experimental.pallas import tpu as pltpu
from jax.experimental.pallas import tpu_sc as plsc

B, D, E = 32768, 768, 64
BLK = 2048

NC, NS = 2, 16
NW = NC * NS
BPW = B // NW
WORDS = BPW * E


def _sumsq_rows(v):
    r = v.shape[0]
    v2 = v * v
    cs = [v2[:, c * 128:(c + 1) * 128] for c in range(6)]
    p = ((cs[0] + cs[1]) + (cs[2] + cs[3])) + (cs[4] + cs[5])
    acc = p[:, 0:8]
    for j in range(1, 16):
        acc = acc + p[:, 8 * j:8 * j + 8]
    acc = acc[:, :4] + acc[:, 4:]
    acc = acc[:, :2] + acc[:, 2:]
    return acc[:, :1] + acc[:, 1:]


def _l2n(v):
    n = jnp.sqrt(_sumsq_rows(v))
    return v / jnp.maximum(n, 1e-12)


def _route_tc(x_ref, w_ref, idx_ref):
    nx = _l2n(x_ref[...]).astype(jnp.bfloat16)
    nw = _l2n(w_ref[...]).astype(jnp.bfloat16)
    sim = jax.lax.dot_general(
        nx, nw,
        dimension_numbers=(((1,), (1,)), ((), ())),
        preferred_element_type=jnp.float32,
    )
    idx_ref[...] = jnp.argmax(sim, axis=1).astype(jnp.int32)[:, None]


@functools.partial(
    pl.kernel,
    out_type=jax.ShapeDtypeStruct((B * E,), jnp.float32),
    mesh=plsc.VectorSubcoreMesh(
        core_axis_name="c", subcore_axis_name="s",
        num_cores=NC, num_subcores=NS,
    ),
    scratch_types=[
        pltpu.VMEM((BPW,), jnp.int32),
        pltpu.VMEM((WORDS,), jnp.float32),
    ],
    compiler_params=pltpu.CompilerParams(needs_layout_passes=False),
)
def _onehot_sc(idx_hbm, out_hbm, idx_v, buf_v):
    wid = lax.axis_index("s") * NC + lax.axis_index("c")
    base = wid * BPW
    pltpu.sync_copy(idx_hbm.at[pl.ds(base, BPW)], idx_v)

    zeros = jnp.zeros((16,), jnp.float32)

    @plsc.parallel_loop(0, WORDS // 16, unroll=8)
    def _zero(i):
        buf_v[pl.ds(i * 16, 16)] = zeros

    ones = jnp.ones((16,), jnp.float32)
    lanes = lax.iota(jnp.int32, 16)

    @plsc.parallel_loop(0, BPW // 16, unroll=4)
    def _scatter(g):
        tok_idx = idx_v[pl.ds(g * 16, 16)]
        pos = (g * 16 + lanes) * E + tok_idx
        plsc.store_scatter(buf_v, [pos], ones)

    pltpu.sync_copy(buf_v, out_hbm.at[pl.ds(base * E, WORDS)])


@jax.jit
def kernel(language_token, routing_embeddings):
    idx = pl.pallas_call(
        _route_tc,
        grid=(B // BLK,),
        in_specs=[
            pl.BlockSpec((BLK, D), lambda i: (i, 0)),
            pl.BlockSpec((E, D), lambda i: (0, 0)),
        ],
        out_specs=pl.BlockSpec((BLK, 1), lambda i: (i, 0)),
        out_shape=jax.ShapeDtypeStruct((B, 1), jnp.int32),
    )(language_token, routing_embeddings)
    weights = _onehot_sc(idx.reshape(B)).reshape(B, E)
    return (weights, idx)


# speedup vs baseline: 1.6803x; 1.6803x over previous
import functools

import jax
import jax.numpy as jnp
from jax import lax
from jax.experimental import pallas as pl
from jax.experimental.pallas import tpu as pltpu
from jax.experimental.pallas import tpu_sc as plsc

B, D, E = 32768, 768, 64
BLK = 2048

NC, NS = 2, 16
NW = NC * NS
BPW = B // NW
WORDS = BPW * E


def _sumsq_rows(v):
    r = v.shape[0]
    v2 = v * v
    cs = [v2[:, c * 128:(c + 1) * 128] for c in range(6)]
    p = ((cs[0] + cs[1]) + (cs[2] + cs[3])) + (cs[4] + cs[5])
    pt = p.T
    acc = pt[0:8, :]
    for j in range(1, 16):
        acc = acc + pt[8 * j:8 * j + 8, :]
    t = acc[0:4, :] + acc[4:8, :]
    t = t[0:2, :] + t[2:4, :]
    s = t[0:1, :] + t[1:2, :]
    return s.T


def _l2n(v):
    n = jnp.sqrt(_sumsq_rows(v))
    return v / jnp.maximum(n, 1e-12)


def _route_tc(x_ref, w_ref, idx_ref):
    nx = _l2n(x_ref[...]).astype(jnp.bfloat16)
    nw = _l2n(w_ref[...]).astype(jnp.bfloat16)
    sim = jax.lax.dot_general(
        nx, nw,
        dimension_numbers=(((1,), (1,)), ((), ())),
        preferred_element_type=jnp.float32,
    )
    idx_ref[...] = jnp.argmax(sim, axis=1).astype(jnp.int32)[:, None]


@functools.partial(
    pl.kernel,
    out_type=jax.ShapeDtypeStruct((B * E,), jnp.float32),
    mesh=plsc.VectorSubcoreMesh(
        core_axis_name="c", subcore_axis_name="s",
        num_cores=NC, num_subcores=NS,
    ),
    scratch_types=[
        pltpu.VMEM((BPW,), jnp.int32),
        pltpu.VMEM((WORDS,), jnp.float32),
    ],
    compiler_params=pltpu.CompilerParams(needs_layout_passes=False),
)
def _onehot_sc(idx_hbm, out_hbm, idx_v, buf_v):
    wid = lax.axis_index("s") * NC + lax.axis_index("c")
    base = wid * BPW
    pltpu.sync_copy(idx_hbm.at[pl.ds(base, BPW)], idx_v)

    zeros = jnp.zeros((16,), jnp.float32)

    @plsc.parallel_loop(0, WORDS // 16, unroll=8)
    def _zero(i):
        buf_v[pl.ds(i * 16, 16)] = zeros

    ones = jnp.ones((16,), jnp.float32)
    lanes = lax.iota(jnp.int32, 16)

    @plsc.parallel_loop(0, BPW // 16, unroll=4)
    def _scatter(g):
        tok_idx = idx_v[pl.ds(g * 16, 16)]
        pos = (g * 16 + lanes) * E + tok_idx
        plsc.store_scatter(buf_v, [pos], ones)

    pltpu.sync_copy(buf_v, out_hbm.at[pl.ds(base * E, WORDS)])


@jax.jit
def kernel(language_token, routing_embeddings):
    idx = pl.pallas_call(
        _route_tc,
        grid=(B // BLK,),
        in_specs=[
            pl.BlockSpec((BLK, D), lambda i: (i, 0)),
            pl.BlockSpec((E, D), lambda i: (0, 0)),
        ],
        out_specs=pl.BlockSpec((BLK, 1), lambda i: (i, 0)),
        out_shape=jax.ShapeDtypeStruct((B, 1), jnp.int32),
    )(language_token, routing_embeddings)
    weights = _onehot_sc(idx.reshape(B)).reshape(B, E)
    return (weights, idx)


# speedup vs baseline: 1.7893x; 1.0649x over previous
import functools

import jax
import jax.numpy as jnp
from jax import lax
from jax.experimental import pallas as pl
from jax.experimental.pallas import tpu as pltpu
from jax.experimental.pallas import tpu_sc as plsc

B, D, E = 32768, 768, 64
BLK = 2048

NC, NS = 2, 16
NW = NC * NS
BPW = B // NW
WORDS = BPW * E


def _sumsq_rows(v):
    r = v.shape[0]
    v2 = v * v
    cs = [v2[:, c * 128:(c + 1) * 128] for c in range(6)]
    p = ((cs[0] + cs[1]) + (cs[2] + cs[3])) + (cs[4] + cs[5])
    pt = p.T
    acc = pt[0:8, :]
    for j in range(1, 16):
        acc = acc + pt[8 * j:8 * j + 8, :]
    t = acc[0:4, :] + acc[4:8, :]
    t = t[0:2, :] + t[2:4, :]
    s = t[0:1, :] + t[1:2, :]
    return s.T


def _l2n(v):
    n = jnp.sqrt(_sumsq_rows(v))
    return v / jnp.maximum(n, 1e-12)


def _route_tc(x_ref, w_ref, idx_ref, idxf_ref):
    nx = _l2n(x_ref[...]).astype(jnp.bfloat16)
    nw = _l2n(w_ref[...]).astype(jnp.bfloat16)
    sim = jax.lax.dot_general(
        nx, nw,
        dimension_numbers=(((1,), (1,)), ((), ())),
        preferred_element_type=jnp.float32,
    )
    idx = jnp.argmax(sim, axis=1).astype(jnp.int32)
    idx_ref[...] = idx[:, None]
    idxf_ref[...] = idx


@functools.partial(
    pl.kernel,
    out_type=jax.ShapeDtypeStruct((B * E,), jnp.float32),
    mesh=plsc.VectorSubcoreMesh(
        core_axis_name="c", subcore_axis_name="s",
        num_cores=NC, num_subcores=NS,
    ),
    scratch_types=[
        pltpu.VMEM((BPW,), jnp.int32),
        pltpu.VMEM((WORDS,), jnp.float32),
    ],
    compiler_params=pltpu.CompilerParams(needs_layout_passes=False),
)
def _onehot_sc(idx_hbm, out_hbm, idx_v, buf_v):
    wid = lax.axis_index("s") * NC + lax.axis_index("c")
    base = wid * BPW
    pltpu.sync_copy(idx_hbm.at[pl.ds(base, BPW)], idx_v)

    zeros = jnp.zeros((16,), jnp.float32)

    @plsc.parallel_loop(0, WORDS // 16, unroll=8)
    def _zero(i):
        buf_v[pl.ds(i * 16, 16)] = zeros

    ones = jnp.ones((16,), jnp.float32)
    lanes = lax.iota(jnp.int32, 16)

    @plsc.parallel_loop(0, BPW // 16, unroll=4)
    def _scatter(g):
        tok_idx = idx_v[pl.ds(g * 16, 16)]
        pos = (g * 16 + lanes) * E + tok_idx
        plsc.store_scatter(buf_v, [pos], ones)

    pltpu.sync_copy(buf_v, out_hbm.at[pl.ds(base * E, WORDS)])


@jax.jit
def kernel(language_token, routing_embeddings):
    idx, idx_flat = pl.pallas_call(
        _route_tc,
        grid=(B // BLK,),
        in_specs=[
            pl.BlockSpec((BLK, D), lambda i: (i, 0)),
            pl.BlockSpec((E, D), lambda i: (0, 0)),
        ],
        out_specs=[
            pl.BlockSpec((BLK, 1), lambda i: (i, 0)),
            pl.BlockSpec((BLK,), lambda i: (i,)),
        ],
        out_shape=[
            jax.ShapeDtypeStruct((B, 1), jnp.int32),
            jax.ShapeDtypeStruct((B,), jnp.int32),
        ],
    )(language_token, routing_embeddings)
    weights = _onehot_sc(idx_flat).reshape(B, E)
    return (weights, idx)


# speedup vs baseline: 2.3002x; 1.2855x over previous
import functools

import jax
import jax.numpy as jnp
from jax import lax
from jax.experimental import pallas as pl
from jax.experimental.pallas import tpu as pltpu
from jax.experimental.pallas import tpu_sc as plsc

B, D, E = 32768, 768, 64
BLK = 2048

NC, NS = 2, 16
NW = NC * NS
BPW = B // NW
WORDS = BPW * E


def _sumsq_rows(v):
    r = v.shape[0]
    v2 = v * v
    cs = [v2[:, c * 128:(c + 1) * 128] for c in range(6)]
    p = ((cs[0] + cs[1]) + (cs[2] + cs[3])) + (cs[4] + cs[5])
    pt = p.T
    acc = pt[0:8, :]
    for j in range(1, 16):
        acc = acc + pt[8 * j:8 * j + 8, :]
    t = acc[0:4, :] + acc[4:8, :]
    t = t[0:2, :] + t[2:4, :]
    s = t[0:1, :] + t[1:2, :]
    return s.T


def _l2n(v):
    n = jnp.sqrt(_sumsq_rows(v))
    return v / jnp.maximum(n, 1e-12)


def _route_tc(x_ref, w_ref, idx_ref, idxf_ref):
    nx = _l2n(x_ref[...]).astype(jnp.bfloat16)
    nw = _l2n(w_ref[...]).astype(jnp.bfloat16)
    sim = jax.lax.dot_general(
        nx, nw,
        dimension_numbers=(((1,), (1,)), ((), ())),
        preferred_element_type=jnp.float32,
    )
    idx = jnp.argmax(sim, axis=1).astype(jnp.int32)
    idx_ref[...] = idx[:, None]
    idxf_ref[...] = idx


@functools.partial(
    pl.kernel,
    out_type=jax.ShapeDtypeStruct((E, B), jnp.float32),
    mesh=plsc.VectorSubcoreMesh(
        core_axis_name="c", subcore_axis_name="s",
        num_cores=NC, num_subcores=NS,
    ),
    scratch_types=[
        pltpu.VMEM((BPW,), jnp.int32),
        pltpu.VMEM((E, BPW), jnp.float32),
    ],
    compiler_params=pltpu.CompilerParams(needs_layout_passes=False),
)
def _onehot_sc(idx_hbm, out_hbm, idx_v, buf_v):
    wid = lax.axis_index("s") * NC + lax.axis_index("c")
    base = wid * BPW
    pltpu.sync_copy(idx_hbm.at[pl.ds(base, BPW)], idx_v)

    zeros = jnp.zeros((16,), jnp.float32)
    ones = jnp.ones((16,), jnp.float32)
    lanes = lax.iota(jnp.int32, 16)

    @plsc.parallel_loop(0, WORDS // 16, unroll=8)
    def _zero(i):
        flat = i * 16 + lanes
        plsc.store_scatter(buf_v, [flat // BPW, flat % BPW], zeros)

    @plsc.parallel_loop(0, BPW // 16, unroll=4)
    def _scatter(g):
        tok = g * 16 + lanes
        tok_idx = idx_v[pl.ds(g * 16, 16)]
        plsc.store_scatter(buf_v, [tok_idx, tok], ones)

    pltpu.sync_copy(buf_v, out_hbm.at[:, pl.ds(base, BPW)])


@jax.jit
def kernel(language_token, routing_embeddings):
    idx, idx_flat = pl.pallas_call(
        _route_tc,
        grid=(B // BLK,),
        in_specs=[
            pl.BlockSpec((BLK, D), lambda i: (i, 0)),
            pl.BlockSpec((E, D), lambda i: (0, 0)),
        ],
        out_specs=[
            pl.BlockSpec((BLK, 1), lambda i: (i, 0)),
            pl.BlockSpec((BLK,), lambda i: (i,)),
        ],
        out_shape=[
            jax.ShapeDtypeStruct((B, 1), jnp.int32),
            jax.ShapeDtypeStruct((B,), jnp.int32),
        ],
    )(language_token, routing_embeddings)
    weights = _onehot_sc(idx_flat).T
    return (weights, idx)
